# baseline (speedup 1.0000x reference)
import functools

import jax
import jax.numpy as jnp
from jax import lax
from jax.experimental import pallas as pl
from jax.experimental.pallas import tpu as pltpu
from jax.experimental.pallas import tpu_sc as plsc

NUM_ROWS = 16384
NUM_INPUTS = 1024
NUM_LEAVES = 256
L = 16
NC, NS = 2, 16
NW = NC * NS
SC_ROWS = 1024
ROWS_PER_W = SC_ROWS // NW
NCHUNK = 2
CHUNK = ROWS_PER_W // NCHUNK
LEAVES_PER_S = NUM_LEAVES // NS
KGRP = NUM_LEAVES // L
TC_BLOCK = 512
TC_BLOCKS = (NUM_ROWS - SC_ROWS) // TC_BLOCK
TC_OFF = SC_ROWS // TC_BLOCK

_mesh = plsc.VectorSubcoreMesh(core_axis_name="c", subcore_axis_name="s")


@functools.partial(
    pl.kernel,
    mesh=_mesh,
    out_type=jax.ShapeDtypeStruct((SC_ROWS, NUM_LEAVES), jnp.float32),
    compiler_params=pltpu.CompilerParams(needs_layout_passes=False,
                                         skip_device_barrier=True),
    scratch_types=[
        pltpu.VMEM((CHUNK, NUM_INPUTS), jnp.float32),
        pltpu.VMEM((CHUNK, NUM_INPUTS), jnp.float32),
        pltpu.VMEM((CHUNK, NUM_LEAVES), jnp.float32),
        pltpu.VMEM((CHUNK, NUM_LEAVES), jnp.float32),
        pltpu.VMEM((LEAVES_PER_S, NUM_INPUTS), jnp.float32),
        pltpu.VMEM((L,), jnp.float32),
        pltpu.VMEM((L,), jnp.int32),
        pltpu.VMEM((NUM_LEAVES,), jnp.int32),
        pltpu.VMEM_SHARED((NUM_LEAVES,), jnp.int32),
        pltpu.SemaphoreType.DMA,
        pltpu.SemaphoreType.DMA,
        pltpu.SemaphoreType.DMA,
        pltpu.SemaphoreType.DMA,
    ],
)
def _frozen_gather(x_hbm, p_hbm, out_hbm,
                   x_v0, x_v1, o_v0, o_v1, p_v, acc_v, loc_v, idx_v,
                   idx_sh, isem0, isem1, osem0, osem1):
    cid = lax.axis_index("c")
    sid = lax.axis_index("s")
    wid = sid * NC + cid
    base = wid * ROWS_PER_W

    def in_slice(g):
        return x_hbm.at[pl.ds(base + g * CHUNK, CHUNK)]

    def out_slice(g):
        return out_hbm.at[pl.ds(base + g * CHUNK, CHUNK)]

    pltpu.async_copy(in_slice(0), x_v0, isem0)
    pltpu.async_copy(in_slice(1), x_v1, isem1)

    pltpu.sync_copy(p_hbm.at[pl.ds(sid * LEAVES_PER_S, LEAVES_PER_S)], p_v)
    lane = lax.iota(jnp.int32, L)
    lane_f = lane.astype(jnp.float32)
    result = jnp.zeros((L,), jnp.int32)
    for leaf in range(LEAVES_PER_S):
        acc = jnp.zeros((L,), jnp.float32)
        for c in range(NUM_INPUTS // L):
            acc = acc + p_v[leaf, pl.ds(c * L, L)] * (lane_f + float(c * L + 1))
        hit = plsc.all_reduce_ffs(acc > 0.5)
        acc_v[...] = acc
        val = plsc.load_gather(acc_v, [hit]) - 1.0
        result = jnp.where(lane == leaf, val.astype(jnp.int32), result)
    loc_v[...] = result
    pltpu.sync_copy(loc_v, idx_sh.at[pl.ds(sid * LEAVES_PER_S, LEAVES_PER_S)])
    plsc.subcore_barrier()
    pltpu.sync_copy(idx_sh, idx_v)
    cols = [idx_v[pl.ds(k * L, L)] for k in range(KGRP)]

    def compute_chunk(x_v, o_v):
        def row_body(r, carry):
            rows = jnp.full((L,), r, jnp.int32)
            for k0 in (0, 8):
                vals = [plsc.load_gather(x_v, [rows, cols[k0 + k]])
                        for k in range(8)]
                for k in range(8):
                    o_v[r, pl.ds((k0 + k) * L, L)] = vals[k]
            return carry
        lax.fori_loop(0, CHUNK, row_body, 0)

    bufs = ((x_v0, o_v0, isem0, osem0), (x_v1, o_v1, isem1, osem1))

    def g2_body(g2, carry):
        for b, (x_v, o_v, isem, osem) in enumerate(bufs):
            g = 2 * g2 + b
            pltpu.make_async_copy(in_slice(g), x_v, isem).wait()

            @pl.when(g2 > 0)
            def _wait_prev_out():
                pltpu.make_async_copy(o_v, out_slice(g - 2), osem).wait()

            compute_chunk(x_v, o_v)
            pltpu.async_copy(o_v, out_slice(g), osem)

            @pl.when(g2 < NCHUNK // 2 - 1)
            def _start_next_in():
                pltpu.async_copy(in_slice(g + 2), x_v, isem)
        return carry

    lax.fori_loop(0, NCHUNK // 2, g2_body, 0)
    pltpu.make_async_copy(o_v0, out_slice(NCHUNK - 2), osem0).wait()
    pltpu.make_async_copy(o_v1, out_slice(NCHUNK - 1), osem1).wait()


def _tc_body(x_ref, pt_ref, o_ref):
    o_ref[...] = jax.lax.dot_general(
        x_ref[...].astype(jnp.bfloat16), pt_ref[...],
        (((1,), (0,)), ((), ())), preferred_element_type=jnp.float32)


_tc_matmul = pl.pallas_call(
    _tc_body,
    grid=(TC_BLOCKS,),
    compiler_params=pltpu.CompilerParams(skip_device_barrier=True),
    in_specs=[
        pl.BlockSpec((TC_BLOCK, NUM_INPUTS), lambda i: (i + TC_OFF, 0)),
        pl.BlockSpec((NUM_INPUTS, NUM_LEAVES), lambda i: (0, 0)),
    ],
    out_specs=pl.BlockSpec((TC_BLOCK, NUM_LEAVES), lambda i: (i + TC_OFF, 0)),
    out_shape=jax.ShapeDtypeStruct((NUM_ROWS, NUM_LEAVES), jnp.float32),
)


def kernel(x, P_hard):
    sc = _frozen_gather(x, P_hard)
    pt = P_hard.T.astype(jnp.bfloat16)
    full = _tc_matmul(x, pt)
    return lax.dynamic_update_slice(full, sc, (0, 0))


# speedup vs baseline: 1.1382x; 1.1382x over previous
import functools

import jax
import jax.numpy as jnp
from jax import lax
from jax.experimental import pallas as pl
from jax.experimental.pallas import tpu as pltpu
from jax.experimental.pallas import tpu_sc as plsc

NUM_ROWS = 16384
NUM_INPUTS = 1024
NUM_LEAVES = 256
L = 16
NC, NS = 2, 16
NW = NC * NS
SC_ROWS = 1024
ROWS_PER_W = SC_ROWS // NW
NCHUNK = 2
CHUNK = ROWS_PER_W // NCHUNK
LEAVES_PER_S = NUM_LEAVES // NS
KGRP = NUM_LEAVES // L
TC_BLOCK = 1024
TC_BLOCKS = (NUM_ROWS - SC_ROWS) // TC_BLOCK
TC_OFF = SC_ROWS // TC_BLOCK

_mesh = plsc.VectorSubcoreMesh(core_axis_name="c", subcore_axis_name="s")


@functools.partial(
    pl.kernel,
    mesh=_mesh,
    out_type=jax.ShapeDtypeStruct((SC_ROWS, NUM_LEAVES), jnp.float32),
    compiler_params=pltpu.CompilerParams(needs_layout_passes=False,
                                         skip_device_barrier=True),
    scratch_types=[
        pltpu.VMEM((CHUNK, NUM_INPUTS), jnp.float32),
        pltpu.VMEM((CHUNK, NUM_INPUTS), jnp.float32),
        pltpu.VMEM((CHUNK, NUM_LEAVES), jnp.float32),
        pltpu.VMEM((CHUNK, NUM_LEAVES), jnp.float32),
        pltpu.VMEM((LEAVES_PER_S, NUM_INPUTS), jnp.float32),
        pltpu.VMEM((L,), jnp.float32),
        pltpu.VMEM((L,), jnp.int32),
        pltpu.VMEM((NUM_LEAVES,), jnp.int32),
        pltpu.VMEM_SHARED((NUM_LEAVES,), jnp.int32),
        pltpu.SemaphoreType.DMA,
        pltpu.SemaphoreType.DMA,
        pltpu.SemaphoreType.DMA,
        pltpu.SemaphoreType.DMA,
    ],
)
def _frozen_gather(x_hbm, p_hbm, out_hbm,
                   x_v0, x_v1, o_v0, o_v1, p_v, acc_v, loc_v, idx_v,
                   idx_sh, isem0, isem1, osem0, osem1):
    cid = lax.axis_index("c")
    sid = lax.axis_index("s")
    wid = sid * NC + cid
    base = wid * ROWS_PER_W

    def in_slice(g):
        return x_hbm.at[pl.ds(base + g * CHUNK, CHUNK)]

    def out_slice(g):
        return out_hbm.at[pl.ds(base + g * CHUNK, CHUNK)]

    pltpu.async_copy(in_slice(0), x_v0, isem0)
    pltpu.async_copy(in_slice(1), x_v1, isem1)

    pltpu.sync_copy(p_hbm.at[pl.ds(sid * LEAVES_PER_S, LEAVES_PER_S)], p_v)
    lane = lax.iota(jnp.int32, L)
    lane_f = lane.astype(jnp.float32)
    result = jnp.zeros((L,), jnp.int32)
    for leaf in range(LEAVES_PER_S):
        acc = jnp.zeros((L,), jnp.float32)
        for c in range(NUM_INPUTS // L):
            acc = acc + p_v[leaf, pl.ds(c * L, L)] * (lane_f + float(c * L + 1))
        hit = plsc.all_reduce_ffs(acc > 0.5)
        acc_v[...] = acc
        val = plsc.load_gather(acc_v, [hit]) - 1.0
        result = jnp.where(lane == leaf, val.astype(jnp.int32), result)
    loc_v[...] = result
    pltpu.sync_copy(loc_v, idx_sh.at[pl.ds(sid * LEAVES_PER_S, LEAVES_PER_S)])
    plsc.subcore_barrier()
    pltpu.sync_copy(idx_sh, idx_v)
    cols = [idx_v[pl.ds(k * L, L)] for k in range(KGRP)]

    def compute_chunk(x_v, o_v):
        def row_body(r, carry):
            rows = jnp.full((L,), r, jnp.int32)
            for k0 in (0, 8):
                vals = [plsc.load_gather(x_v, [rows, cols[k0 + k]])
                        for k in range(8)]
                for k in range(8):
                    o_v[r, pl.ds((k0 + k) * L, L)] = vals[k]
            return carry
        lax.fori_loop(0, CHUNK, row_body, 0)

    bufs = ((x_v0, o_v0, isem0, osem0), (x_v1, o_v1, isem1, osem1))

    def g2_body(g2, carry):
        for b, (x_v, o_v, isem, osem) in enumerate(bufs):
            g = 2 * g2 + b
            pltpu.make_async_copy(in_slice(g), x_v, isem).wait()

            @pl.when(g2 > 0)
            def _wait_prev_out():
                pltpu.make_async_copy(o_v, out_slice(g - 2), osem).wait()

            compute_chunk(x_v, o_v)
            pltpu.async_copy(o_v, out_slice(g), osem)

            @pl.when(g2 < NCHUNK // 2 - 1)
            def _start_next_in():
                pltpu.async_copy(in_slice(g + 2), x_v, isem)
        return carry

    lax.fori_loop(0, NCHUNK // 2, g2_body, 0)
    pltpu.make_async_copy(o_v0, out_slice(NCHUNK - 2), osem0).wait()
    pltpu.make_async_copy(o_v1, out_slice(NCHUNK - 1), osem1).wait()


def _tc_body(x_ref, pt_ref, o_ref):
    o_ref[...] = jax.lax.dot_general(
        x_ref[...].astype(jnp.bfloat16), pt_ref[...],
        (((1,), (0,)), ((), ())), preferred_element_type=jnp.float32)


_tc_matmul = pl.pallas_call(
    _tc_body,
    grid=(TC_BLOCKS,),
    compiler_params=pltpu.CompilerParams(skip_device_barrier=True),
    in_specs=[
        pl.BlockSpec((TC_BLOCK, NUM_INPUTS), lambda i: (i + TC_OFF, 0)),
        pl.BlockSpec((NUM_INPUTS, NUM_LEAVES), lambda i: (0, 0)),
    ],
    out_specs=pl.BlockSpec((TC_BLOCK, NUM_LEAVES), lambda i: (i + TC_OFF, 0)),
    out_shape=jax.ShapeDtypeStruct((NUM_ROWS, NUM_LEAVES), jnp.float32),
)


def kernel(x, P_hard):
    sc = _frozen_gather(x, P_hard)
    pt = P_hard.T.astype(jnp.bfloat16)
    full = _tc_matmul(x, pt)
    return lax.dynamic_update_slice(full, sc, (0, 0))


# speedup vs baseline: 1.1800x; 1.0367x over previous
import functools

import jax
import jax.numpy as jnp
from jax import lax
from jax.experimental import pallas as pl
from jax.experimental.pallas import tpu as pltpu
from jax.experimental.pallas import tpu_sc as plsc

NUM_ROWS = 16384
NUM_INPUTS = 1024
NUM_LEAVES = 256
L = 16
NC, NS = 2, 16
NW = NC * NS
SC_ROWS = 2048
ROWS_PER_W = SC_ROWS // NW
NCHUNK = 4
CHUNK = ROWS_PER_W // NCHUNK
LEAVES_PER_S = NUM_LEAVES // NS
KGRP = NUM_LEAVES // L
TC_BLOCK = 2048
TC_BLOCKS = (NUM_ROWS - SC_ROWS) // TC_BLOCK
TC_OFF = SC_ROWS // TC_BLOCK

_mesh = plsc.VectorSubcoreMesh(core_axis_name="c", subcore_axis_name="s")


@functools.partial(
    pl.kernel,
    mesh=_mesh,
    out_type=jax.ShapeDtypeStruct((SC_ROWS, NUM_LEAVES), jnp.float32),
    compiler_params=pltpu.CompilerParams(needs_layout_passes=False,
                                         skip_device_barrier=True),
    scratch_types=[
        pltpu.VMEM((CHUNK, NUM_INPUTS), jnp.float32),
        pltpu.VMEM((CHUNK, NUM_INPUTS), jnp.float32),
        pltpu.VMEM((CHUNK, NUM_LEAVES), jnp.float32),
        pltpu.VMEM((CHUNK, NUM_LEAVES), jnp.float32),
        pltpu.VMEM((LEAVES_PER_S, NUM_INPUTS), jnp.float32),
        pltpu.VMEM((L,), jnp.float32),
        pltpu.VMEM((L,), jnp.int32),
        pltpu.VMEM((NUM_LEAVES,), jnp.int32),
        pltpu.VMEM_SHARED((NUM_LEAVES,), jnp.int32),
        pltpu.SemaphoreType.DMA,
        pltpu.SemaphoreType.DMA,
        pltpu.SemaphoreType.DMA,
        pltpu.SemaphoreType.DMA,
    ],
)
def _frozen_gather(x_hbm, p_hbm, out_hbm,
                   x_v0, x_v1, o_v0, o_v1, p_v, acc_v, loc_v, idx_v,
                   idx_sh, isem0, isem1, osem0, osem1):
    cid = lax.axis_index("c")
    sid = lax.axis_index("s")
    wid = sid * NC + cid
    base = wid * ROWS_PER_W

    def in_slice(g):
        return x_hbm.at[pl.ds(base + g * CHUNK, CHUNK)]

    def out_slice(g):
        return out_hbm.at[pl.ds(base + g * CHUNK, CHUNK)]

    pltpu.async_copy(in_slice(0), x_v0, isem0)
    pltpu.async_copy(in_slice(1), x_v1, isem1)

    pltpu.sync_copy(p_hbm.at[pl.ds(sid * LEAVES_PER_S, LEAVES_PER_S)], p_v)
    lane = lax.iota(jnp.int32, L)
    lane_f = lane.astype(jnp.float32)
    result = jnp.zeros((L,), jnp.int32)
    for leaf in range(LEAVES_PER_S):
        acc = jnp.zeros((L,), jnp.float32)
        for c in range(NUM_INPUTS // L):
            acc = acc + p_v[leaf, pl.ds(c * L, L)] * (lane_f + float(c * L + 1))
        hit = plsc.all_reduce_ffs(acc > 0.5)
        acc_v[...] = acc
        val = plsc.load_gather(acc_v, [hit]) - 1.0
        result = jnp.where(lane == leaf, val.astype(jnp.int32), result)
    loc_v[...] = result
    pltpu.sync_copy(loc_v, idx_sh.at[pl.ds(sid * LEAVES_PER_S, LEAVES_PER_S)])
    plsc.subcore_barrier()
    pltpu.sync_copy(idx_sh, idx_v)
    cols = [idx_v[pl.ds(k * L, L)] for k in range(KGRP)]

    def compute_chunk(x_v, o_v):
        def row_body(r, carry):
            rows = jnp.full((L,), r, jnp.int32)
            for k0 in (0, 8):
                vals = [plsc.load_gather(x_v, [rows, cols[k0 + k]])
                        for k in range(8)]
                for k in range(8):
                    o_v[r, pl.ds((k0 + k) * L, L)] = vals[k]
            return carry
        lax.fori_loop(0, CHUNK, row_body, 0)

    bufs = ((x_v0, o_v0, isem0, osem0), (x_v1, o_v1, isem1, osem1))

    def g2_body(g2, carry):
        for b, (x_v, o_v, isem, osem) in enumerate(bufs):
            g = 2 * g2 + b
            pltpu.make_async_copy(in_slice(g), x_v, isem).wait()

            @pl.when(g2 > 0)
            def _wait_prev_out():
                pltpu.make_async_copy(o_v, out_slice(g - 2), osem).wait()

            compute_chunk(x_v, o_v)
            pltpu.async_copy(o_v, out_slice(g), osem)

            @pl.when(g2 < NCHUNK // 2 - 1)
            def _start_next_in():
                pltpu.async_copy(in_slice(g + 2), x_v, isem)
        return carry

    lax.fori_loop(0, NCHUNK // 2, g2_body, 0)
    pltpu.make_async_copy(o_v0, out_slice(NCHUNK - 2), osem0).wait()
    pltpu.make_async_copy(o_v1, out_slice(NCHUNK - 1), osem1).wait()


def _tc_body(x_ref, pt_ref, o_ref):
    o_ref[...] = jax.lax.dot_general(
        x_ref[...].astype(jnp.bfloat16), pt_ref[...],
        (((1,), (0,)), ((), ())), preferred_element_type=jnp.float32)


_tc_matmul = pl.pallas_call(
    _tc_body,
    grid=(TC_BLOCKS,),
    compiler_params=pltpu.CompilerParams(skip_device_barrier=True),
    in_specs=[
        pl.BlockSpec((TC_BLOCK, NUM_INPUTS), lambda i: (i + TC_OFF, 0)),
        pl.BlockSpec((NUM_INPUTS, NUM_LEAVES), lambda i: (0, 0)),
    ],
    out_specs=pl.BlockSpec((TC_BLOCK, NUM_LEAVES), lambda i: (i + TC_OFF, 0)),
    out_shape=jax.ShapeDtypeStruct((NUM_ROWS, NUM_LEAVES), jnp.float32),
)


def kernel(x, P_hard):
    sc = _frozen_gather(x, P_hard)
    pt = P_hard.T.astype(jnp.bfloat16)
    full = _tc_matmul(x, pt)
    return lax.dynamic_update_slice(full, sc, (0, 0))


# speedup vs baseline: 1.2059x; 1.0219x over previous
import functools

import jax
import jax.numpy as jnp
from jax import lax
from jax.experimental import pallas as pl
from jax.experimental.pallas import tpu as pltpu
from jax.experimental.pallas import tpu_sc as plsc

NUM_ROWS = 16384
NUM_INPUTS = 1024
NUM_LEAVES = 256
L = 16
NC, NS = 2, 16
NW = NC * NS
SC_ROWS = 2048
ROWS_PER_W = SC_ROWS // NW
NCHUNK = 4
CHUNK = ROWS_PER_W // NCHUNK
LEAVES_PER_S = NUM_LEAVES // NS
KGRP = NUM_LEAVES // L
TC_BLOCK = 2048
TC_BLOCKS = (NUM_ROWS - SC_ROWS) // TC_BLOCK
TC_OFF = SC_ROWS // TC_BLOCK

_mesh = plsc.VectorSubcoreMesh(core_axis_name="c", subcore_axis_name="s")


@functools.partial(
    pl.kernel,
    mesh=_mesh,
    out_type=jax.ShapeDtypeStruct((SC_ROWS, NUM_LEAVES), jnp.float32),
    compiler_params=pltpu.CompilerParams(needs_layout_passes=False,
                                         skip_device_barrier=True),
    scratch_types=[
        pltpu.VMEM((CHUNK, NUM_INPUTS), jnp.float32),
        pltpu.VMEM((CHUNK, NUM_INPUTS), jnp.float32),
        pltpu.VMEM((CHUNK, NUM_LEAVES), jnp.float32),
        pltpu.VMEM((CHUNK, NUM_LEAVES), jnp.float32),
        pltpu.VMEM((LEAVES_PER_S, NUM_INPUTS), jnp.float32),
        pltpu.VMEM((L,), jnp.float32),
        pltpu.VMEM((L,), jnp.int32),
        pltpu.VMEM((NUM_LEAVES,), jnp.int32),
        pltpu.VMEM_SHARED((NUM_LEAVES,), jnp.int32),
        pltpu.SemaphoreType.DMA,
        pltpu.SemaphoreType.DMA,
        pltpu.SemaphoreType.DMA,
        pltpu.SemaphoreType.DMA,
    ],
)
def _frozen_gather(x_hbm, p_hbm, out_hbm,
                   x_v0, x_v1, o_v0, o_v1, p_v, acc_v, loc_v, idx_v,
                   idx_sh, isem0, isem1, osem0, osem1):
    cid = lax.axis_index("c")
    sid = lax.axis_index("s")
    wid = sid * NC + cid
    base = wid * ROWS_PER_W

    def in_slice(g):
        return x_hbm.at[pl.ds(base + g * CHUNK, CHUNK)]

    def out_slice(g):
        return out_hbm.at[pl.ds(base + g * CHUNK, CHUNK)]

    pltpu.async_copy(in_slice(0), x_v0, isem0)
    pltpu.async_copy(in_slice(1), x_v1, isem1)

    pltpu.sync_copy(p_hbm.at[pl.ds(sid * LEAVES_PER_S, LEAVES_PER_S)], p_v)
    lane = lax.iota(jnp.int32, L)
    lane_f = lane.astype(jnp.float32)
    result = jnp.zeros((L,), jnp.int32)
    for leaf in range(LEAVES_PER_S):
        acc = jnp.zeros((L,), jnp.float32)
        for c in range(NUM_INPUTS // L):
            acc = acc + p_v[leaf, pl.ds(c * L, L)] * (lane_f + float(c * L + 1))
        hit = plsc.all_reduce_ffs(acc > 0.5)
        acc_v[...] = acc
        val = plsc.load_gather(acc_v, [hit]) - 1.0
        result = jnp.where(lane == leaf, val.astype(jnp.int32), result)
    loc_v[...] = result
    pltpu.sync_copy(loc_v, idx_sh.at[pl.ds(sid * LEAVES_PER_S, LEAVES_PER_S)])
    plsc.subcore_barrier()
    pltpu.sync_copy(idx_sh, idx_v)
    cols = [idx_v[pl.ds(k * L, L)] for k in range(KGRP)]

    def compute_chunk(x_v, o_v):
        def row_body(r, carry):
            rows = jnp.full((L,), r, jnp.int32)
            for k0 in (0, 8):
                vals = [plsc.load_gather(x_v, [rows, cols[k0 + k]])
                        for k in range(8)]
                for k in range(8):
                    o_v[r, pl.ds((k0 + k) * L, L)] = vals[k]
            return carry
        lax.fori_loop(0, CHUNK, row_body, 0)

    bufs = ((x_v0, o_v0, isem0, osem0), (x_v1, o_v1, isem1, osem1))

    def g2_body(g2, carry):
        for b, (x_v, o_v, isem, osem) in enumerate(bufs):
            g = 2 * g2 + b
            pltpu.make_async_copy(in_slice(g), x_v, isem).wait()

            @pl.when(g2 > 0)
            def _wait_prev_out():
                pltpu.make_async_copy(o_v, out_slice(g - 2), osem).wait()

            compute_chunk(x_v, o_v)
            pltpu.async_copy(o_v, out_slice(g), osem)

            @pl.when(g2 < NCHUNK // 2 - 1)
            def _start_next_in():
                pltpu.async_copy(in_slice(g + 2), x_v, isem)
        return carry

    lax.fori_loop(0, NCHUNK // 2, g2_body, 0)
    pltpu.make_async_copy(o_v0, out_slice(NCHUNK - 2), osem0).wait()
    pltpu.make_async_copy(o_v1, out_slice(NCHUNK - 1), osem1).wait()


NBUF = 4
BR = 512
TC_STEPS = (NUM_ROWS - SC_ROWS) // BR


def _tc_body(x_hbm, pt_ref, o_hbm, xb, ob, isems, osems):
    def in_cp(s, b):
        return pltpu.make_async_copy(
            x_hbm.at[pl.ds(SC_ROWS + s * BR, BR)], xb.at[b], isems.at[b])

    def out_cp(s, b):
        return pltpu.make_async_copy(
            ob.at[b], o_hbm.at[pl.ds(SC_ROWS + s * BR, BR)], osems.at[b])

    for b in range(NBUF):
        in_cp(b, b).start()

    def g_body(g, carry):
        for b in range(NBUF):
            s = g * NBUF + b
            in_cp(s, b).wait()

            @pl.when(g > 0)
            def _wait_prev_out():
                out_cp(s - NBUF, b).wait()

            ob[b] = jax.lax.dot_general(
                xb[b].astype(jnp.bfloat16), pt_ref[...],
                (((1,), (0,)), ((), ())), preferred_element_type=jnp.float32)
            out_cp(s, b).start()

            @pl.when(g < TC_STEPS // NBUF - 1)
            def _start_next_in():
                in_cp(s + NBUF, b).start()
        return carry

    lax.fori_loop(0, TC_STEPS // NBUF, g_body, 0)
    for b in range(NBUF):
        out_cp(TC_STEPS - NBUF + b, b).wait()


_tc_matmul = pl.pallas_call(
    _tc_body,
    compiler_params=pltpu.CompilerParams(skip_device_barrier=True),
    in_specs=[
        pl.BlockSpec(memory_space=pl.ANY),
        pl.BlockSpec((NUM_INPUTS, NUM_LEAVES), lambda: (0, 0)),
    ],
    out_specs=pl.BlockSpec(memory_space=pl.ANY),
    out_shape=jax.ShapeDtypeStruct((NUM_ROWS, NUM_LEAVES), jnp.float32),
    scratch_shapes=[
        pltpu.VMEM((NBUF, BR, NUM_INPUTS), jnp.float32),
        pltpu.VMEM((NBUF, BR, NUM_LEAVES), jnp.float32),
        pltpu.SemaphoreType.DMA((NBUF,)),
        pltpu.SemaphoreType.DMA((NBUF,)),
    ],
)


def kernel(x, P_hard):
    sc = _frozen_gather(x, P_hard)
    pt = P_hard.T.astype(jnp.bfloat16)
    full = _tc_matmul(x, pt)
    return lax.dynamic_update_slice(full, sc, (0, 0))


# speedup vs baseline: 1.2190x; 1.0109x over previous
import functools

import jax
import jax.numpy as jnp
from jax import lax
from jax.experimental import pallas as pl
from jax.experimental.pallas import tpu as pltpu
from jax.experimental.pallas import tpu_sc as plsc

NUM_ROWS = 16384
NUM_INPUTS = 1024
NUM_LEAVES = 256
L = 16
NC, NS = 2, 16
NW = NC * NS
SC_ROWS = 2048
ROWS_PER_W = SC_ROWS // NW
NCHUNK = 4
CHUNK = ROWS_PER_W // NCHUNK
LEAVES_PER_S = NUM_LEAVES // NS
KGRP = NUM_LEAVES // L
TC_BLOCK = 2048
TC_BLOCKS = (NUM_ROWS - SC_ROWS) // TC_BLOCK
TC_OFF = SC_ROWS // TC_BLOCK

_mesh = plsc.VectorSubcoreMesh(core_axis_name="c", subcore_axis_name="s")


@functools.partial(
    pl.kernel,
    mesh=_mesh,
    out_type=jax.ShapeDtypeStruct((SC_ROWS, NUM_LEAVES), jnp.float32),
    compiler_params=pltpu.CompilerParams(needs_layout_passes=False,
                                         skip_device_barrier=True),
    scratch_types=[
        pltpu.VMEM((CHUNK, NUM_INPUTS), jnp.float32),
        pltpu.VMEM((CHUNK, NUM_INPUTS), jnp.float32),
        pltpu.VMEM((CHUNK, NUM_LEAVES), jnp.float32),
        pltpu.VMEM((CHUNK, NUM_LEAVES), jnp.float32),
        pltpu.VMEM((LEAVES_PER_S, NUM_INPUTS), jnp.float32),
        pltpu.VMEM((L,), jnp.float32),
        pltpu.VMEM((L,), jnp.int32),
        pltpu.VMEM((NUM_LEAVES,), jnp.int32),
        pltpu.VMEM_SHARED((NUM_LEAVES,), jnp.int32),
        pltpu.SemaphoreType.DMA,
        pltpu.SemaphoreType.DMA,
        pltpu.SemaphoreType.DMA,
        pltpu.SemaphoreType.DMA,
    ],
)
def _frozen_gather(x_hbm, p_hbm, out_hbm,
                   x_v0, x_v1, o_v0, o_v1, p_v, acc_v, loc_v, idx_v,
                   idx_sh, isem0, isem1, osem0, osem1):
    cid = lax.axis_index("c")
    sid = lax.axis_index("s")
    wid = sid * NC + cid
    base = wid * ROWS_PER_W

    def in_slice(g):
        return x_hbm.at[pl.ds(base + g * CHUNK, CHUNK)]

    def out_slice(g):
        return out_hbm.at[pl.ds(base + g * CHUNK, CHUNK)]

    pltpu.async_copy(in_slice(0), x_v0, isem0)
    pltpu.async_copy(in_slice(1), x_v1, isem1)

    pltpu.sync_copy(p_hbm.at[pl.ds(sid * LEAVES_PER_S, LEAVES_PER_S)], p_v)
    lane = lax.iota(jnp.int32, L)
    lane_f = lane.astype(jnp.float32)
    result = jnp.zeros((L,), jnp.int32)
    for leaf in range(LEAVES_PER_S):
        acc = jnp.zeros((L,), jnp.float32)
        for c in range(NUM_INPUTS // L):
            acc = acc + p_v[leaf, pl.ds(c * L, L)] * (lane_f + float(c * L + 1))
        hit = plsc.all_reduce_ffs(acc > 0.5)
        acc_v[...] = acc
        val = plsc.load_gather(acc_v, [hit]) - 1.0
        result = jnp.where(lane == leaf, val.astype(jnp.int32), result)
    loc_v[...] = result
    pltpu.sync_copy(loc_v, idx_sh.at[pl.ds(sid * LEAVES_PER_S, LEAVES_PER_S)])
    plsc.subcore_barrier()
    pltpu.sync_copy(idx_sh, idx_v)
    cols = [idx_v[pl.ds(k * L, L)] for k in range(KGRP)]

    def compute_chunk(x_v, o_v):
        def row_body(r, carry):
            rows = jnp.full((L,), r, jnp.int32)
            for k0 in (0, 8):
                vals = [plsc.load_gather(x_v, [rows, cols[k0 + k]])
                        for k in range(8)]
                for k in range(8):
                    o_v[r, pl.ds((k0 + k) * L, L)] = vals[k]
            return carry
        lax.fori_loop(0, CHUNK, row_body, 0)

    bufs = ((x_v0, o_v0, isem0, osem0), (x_v1, o_v1, isem1, osem1))

    def g2_body(g2, carry):
        for b, (x_v, o_v, isem, osem) in enumerate(bufs):
            g = 2 * g2 + b
            pltpu.make_async_copy(in_slice(g), x_v, isem).wait()

            @pl.when(g2 > 0)
            def _wait_prev_out():
                pltpu.make_async_copy(o_v, out_slice(g - 2), osem).wait()

            compute_chunk(x_v, o_v)
            pltpu.async_copy(o_v, out_slice(g), osem)

            @pl.when(g2 < NCHUNK // 2 - 1)
            def _start_next_in():
                pltpu.async_copy(in_slice(g + 2), x_v, isem)
        return carry

    lax.fori_loop(0, NCHUNK // 2, g2_body, 0)
    pltpu.make_async_copy(o_v0, out_slice(NCHUNK - 2), osem0).wait()
    pltpu.make_async_copy(o_v1, out_slice(NCHUNK - 1), osem1).wait()


NBUF = 4
BR = 512
TC_STEPS = (NUM_ROWS - SC_ROWS) // BR


def _tc_body(x_hbm, p_ref, o_hbm, xb, ob, pbt, isems, osems):
    pbt[...] = p_ref[...].astype(jnp.bfloat16).T
    def in_cp(s, b):
        return pltpu.make_async_copy(
            x_hbm.at[pl.ds(SC_ROWS + s * BR, BR)], xb.at[b], isems.at[b])

    def out_cp(s, b):
        return pltpu.make_async_copy(
            ob.at[b], o_hbm.at[pl.ds(SC_ROWS + s * BR, BR)], osems.at[b])

    for b in range(NBUF):
        in_cp(b, b).start()

    def g_body(g, carry):
        for b in range(NBUF):
            s = g * NBUF + b
            in_cp(s, b).wait()

            @pl.when(g > 0)
            def _wait_prev_out():
                out_cp(s - NBUF, b).wait()

            ob[b] = jax.lax.dot_general(
                xb[b].astype(jnp.bfloat16), pbt[...],
                (((1,), (0,)), ((), ())), preferred_element_type=jnp.float32)
            out_cp(s, b).start()

            @pl.when(g < TC_STEPS // NBUF - 1)
            def _start_next_in():
                in_cp(s + NBUF, b).start()
        return carry

    lax.fori_loop(0, TC_STEPS // NBUF, g_body, 0)
    for b in range(NBUF):
        out_cp(TC_STEPS - NBUF + b, b).wait()


_tc_matmul = pl.pallas_call(
    _tc_body,
    compiler_params=pltpu.CompilerParams(skip_device_barrier=True),
    in_specs=[
        pl.BlockSpec(memory_space=pl.ANY),
        pl.BlockSpec((NUM_LEAVES, NUM_INPUTS), lambda: (0, 0)),
    ],
    out_specs=pl.BlockSpec(memory_space=pl.ANY),
    out_shape=jax.ShapeDtypeStruct((NUM_ROWS, NUM_LEAVES), jnp.float32),
    scratch_shapes=[
        pltpu.VMEM((NBUF, BR, NUM_INPUTS), jnp.float32),
        pltpu.VMEM((NBUF, BR, NUM_LEAVES), jnp.float32),
        pltpu.VMEM((NUM_INPUTS, NUM_LEAVES), jnp.bfloat16),
        pltpu.SemaphoreType.DMA((NBUF,)),
        pltpu.SemaphoreType.DMA((NBUF,)),
    ],
)


def kernel(x, P_hard):
    sc = _frozen_gather(x, P_hard)
    full = _tc_matmul(x, P_hard)
    return lax.dynamic_update_slice(full, sc, (0, 0))
